# Initial kernel scaffold; baseline (speedup 1.0000x reference)
import jax
import jax.numpy as jnp
from jax.experimental import pallas as pl


def kernel(x, edge_index, W1, b1, W2, b2, Wmu, bmu, Wls, bls):
    raise NotImplementedError("write your pallas kernel here")


# speedup vs baseline: 15.4979x; 15.4979x over previous
import functools

import jax
import jax.numpy as jnp
from jax import lax
from jax.experimental import pallas as pl
from jax.experimental.pallas import tpu as pltpu
from jax.experimental.pallas import tpu_sc as plsc

N = 10000
E = 160000
IN_DIMS = 256
HID1 = 256
HID2 = 128
OUT = 64

NC = 2
NS = 16
G = 125
NCH = E // (NS * G)
NCH_D = E // (NC * NS * G)
NP = 10240
RPT = NP // NS
W128 = 128

_MESH = plsc.VectorSubcoreMesh(core_axis_name="c", subcore_axis_name="s")


def _deg_body(dst_hbm, ones_hbm, zeros_hbm, d0_hbm, d1_hbm, dst_v, ones_v, acc):
    c = lax.axis_index("c")
    s = lax.axis_index("s")
    w = c * NS + s
    pltpu.sync_copy(dst_hbm.at[w], dst_v)
    pltpu.sync_copy(ones_hbm, ones_v)
    r0 = s * RPT
    pltpu.sync_copy(zeros_hbm.at[pl.ds(r0, RPT)], acc.at[pl.ds(r0, RPT)])
    plsc.subcore_barrier()

    def chunk(j, carry):
        pltpu.sync_copy(ones_v, acc.at[dst_v.at[j]], add=True)
        return carry

    lax.fori_loop(0, NCH_D, chunk, 0)
    plsc.subcore_barrier()

    def wout(out):
        pltpu.sync_copy(acc.at[pl.ds(r0, RPT)], out.at[pl.ds(r0, RPT)])

    pl.when(c == 0)(lambda: wout(d0_hbm))
    pl.when(c == 1)(lambda: wout(d1_hbm))


_deg_call = pl.kernel(
    _deg_body,
    out_type=[jax.ShapeDtypeStruct((NP, W128), jnp.float32)] * 2,
    mesh=_MESH,
    scratch_types=[
        pltpu.VMEM((NCH_D, G), jnp.int32),
        pltpu.VMEM((G, W128), jnp.float32),
        pltpu.VMEM_SHARED((NP, W128), jnp.float32),
    ],
)


def _pass_feat_body(hws0, hws1, src_hbm, dst_hbm, out0, out1,
                    src_v, dst_v, buf, acc):
    c = lax.axis_index("c")
    s = lax.axis_index("s")
    pltpu.sync_copy(src_hbm.at[s], src_v)
    pltpu.sync_copy(dst_hbm.at[s], dst_v)
    r0 = s * RPT

    def work(table, out):
        pltpu.sync_copy(table.at[pl.ds(r0, RPT)], acc.at[pl.ds(r0, RPT)])
        plsc.subcore_barrier()

        def chunk(j, carry):
            pltpu.sync_copy(table.at[src_v.at[j]], buf)
            pltpu.sync_copy(buf, acc.at[dst_v.at[j]], add=True)
            return carry

        lax.fori_loop(0, NCH, chunk, 0)
        plsc.subcore_barrier()
        pltpu.sync_copy(acc.at[pl.ds(r0, RPT)], out.at[pl.ds(r0, RPT)])

    pl.when(c == 0)(lambda: work(hws0, out0))
    pl.when(c == 1)(lambda: work(hws1, out1))


_pass_feat_call = pl.kernel(
    _pass_feat_body,
    out_type=[jax.ShapeDtypeStruct((NP, W128), jnp.float32)] * 2,
    mesh=_MESH,
    scratch_types=[
        pltpu.VMEM((NCH, G), jnp.int32),
        pltpu.VMEM((NCH, G), jnp.int32),
        pltpu.VMEM((G, W128), jnp.float32),
        pltpu.VMEM_SHARED((NP, W128), jnp.float32),
    ],
)


def _pass_edge_body(hws, zeros_hbm, src_hbm, dst_hbm, out0, out1,
                    src_v, dst_v, buf, acc):
    c = lax.axis_index("c")
    s = lax.axis_index("s")
    w = c * NS + s
    pltpu.sync_copy(src_hbm.at[w], src_v)
    pltpu.sync_copy(dst_hbm.at[w], dst_v)
    r0 = s * RPT

    def init(src_arr):
        pltpu.sync_copy(src_arr.at[pl.ds(r0, RPT)], acc.at[pl.ds(r0, RPT)])

    pl.when(c == 0)(lambda: init(hws))
    pl.when(c == 1)(lambda: init(zeros_hbm))
    plsc.subcore_barrier()

    def chunk(j, carry):
        pltpu.sync_copy(hws.at[src_v.at[j]], buf)
        pltpu.sync_copy(buf, acc.at[dst_v.at[j]], add=True)
        return carry

    lax.fori_loop(0, NCH_D, chunk, 0)
    plsc.subcore_barrier()

    def wout(out):
        pltpu.sync_copy(acc.at[pl.ds(r0, RPT)], out.at[pl.ds(r0, RPT)])

    pl.when(c == 0)(lambda: wout(out0))
    pl.when(c == 1)(lambda: wout(out1))


_pass_edge_call = pl.kernel(
    _pass_edge_body,
    out_type=[jax.ShapeDtypeStruct((NP, W128), jnp.float32)] * 2,
    mesh=_MESH,
    scratch_types=[
        pltpu.VMEM((NCH_D, G), jnp.int32),
        pltpu.VMEM((NCH_D, G), jnp.int32),
        pltpu.VMEM((G, W128), jnp.float32),
        pltpu.VMEM_SHARED((NP, W128), jnp.float32),
    ],
)


TM = 1000


def _dinv_of(d0, d1):
    deg = d0[:, 0:1] + d1[:, 0:1] + 1.0
    return lax.rsqrt(deg)


def _mm1_body(x_ref, w_ref, b_ref, d0_ref, d1_ref, o0_ref, o1_ref):
    dinv = _dinv_of(d0_ref[...], d1_ref[...])
    hw = jnp.dot(x_ref[...], w_ref[...], preferred_element_type=jnp.float32)
    hws = (hw + b_ref[...]) * dinv
    half = hws.shape[1] // 2
    o0_ref[...] = hws[:, :half]
    o1_ref[...] = hws[:, half:]


def _mm_cat_body(a0_ref, a1_ref, w_ref, b_ref, d0_ref, d1_ref, o_ref):
    dinv = _dinv_of(d0_ref[...], d1_ref[...])
    h = jnp.concatenate([a0_ref[...], a1_ref[...]], axis=1) * dinv
    h = jnp.maximum(h, 0.0)
    hw = jnp.dot(h, w_ref[...], preferred_element_type=jnp.float32)
    o_ref[...] = (hw + b_ref[...]) * dinv


def _mm_add_body(a0_ref, a1_ref, w_ref, b_ref, d0_ref, d1_ref, o_ref):
    dinv = _dinv_of(d0_ref[...], d1_ref[...])
    h = (a0_ref[...] + a1_ref[...]) * dinv
    h = jnp.maximum(h, 0.0)
    hw = jnp.dot(h, w_ref[...], preferred_element_type=jnp.float32)
    o_ref[...] = (hw + b_ref[...]) * dinv


def _z_body(a0_ref, a1_ref, d0_ref, d1_ref, eps_ref, z_ref):
    dinv = _dinv_of(d0_ref[...], d1_ref[...])
    ml = (a0_ref[...] + a1_ref[...]) * dinv
    mu = ml[:, :OUT]
    logstd = jnp.minimum(ml[:, OUT:], 10.0)
    z_ref[...] = mu + eps_ref[...] * jnp.exp(logstd)


def _row_spec(cols):
    return pl.BlockSpec((TM, cols), lambda i: (i, 0))


def _full_spec(shape):
    return pl.BlockSpec(shape, lambda i: (0,) * len(shape))


def _mm1(x, W, b, d0, d1):
    dout = W.shape[1]
    return pl.pallas_call(
        _mm1_body,
        grid=(N // TM,),
        in_specs=[
            _row_spec(x.shape[1]),
            _full_spec(W.shape),
            _full_spec((1, dout)),
            _row_spec(W128),
            _row_spec(W128),
        ],
        out_specs=[_row_spec(dout // 2)] * 2,
        out_shape=[jax.ShapeDtypeStruct((NP, dout // 2), jnp.float32)] * 2,
    )(x, W, b.reshape(1, -1), d0, d1)


def _mm2(body, a0, a1, W, b, d0, d1):
    dout = W.shape[1]
    return pl.pallas_call(
        body,
        grid=(N // TM,),
        in_specs=[
            _row_spec(a0.shape[1]),
            _row_spec(a1.shape[1]),
            _full_spec(W.shape),
            _full_spec((1, dout)),
            _row_spec(W128),
            _row_spec(W128),
        ],
        out_specs=_row_spec(dout),
        out_shape=jax.ShapeDtypeStruct((NP, dout), jnp.float32),
    )(a0, a1, W, b.reshape(1, -1), d0, d1)


def _zfin(a0, a1, d0, d1, eps):
    return pl.pallas_call(
        _z_body,
        grid=(N // TM,),
        in_specs=[
            _row_spec(W128),
            _row_spec(W128),
            _row_spec(W128),
            _row_spec(W128),
            _row_spec(OUT),
        ],
        out_specs=_row_spec(OUT),
        out_shape=jax.ShapeDtypeStruct((N, OUT), jnp.float32),
    )(a0, a1, d0, d1, eps)


def kernel(x, edge_index, W1, b1, W2, b2, Wmu, bmu, Wls, bls):
    src = edge_index[0].astype(jnp.int32)
    dst = edge_index[1].astype(jnp.int32)
    src16 = src.reshape(NS, NCH, G)
    dst16 = dst.reshape(NS, NCH, G)
    src32 = src.reshape(NC * NS, NCH_D, G)
    dst32 = dst.reshape(NC * NS, NCH_D, G)

    ones_g = jnp.ones((G, W128), jnp.float32)
    zeros_n = jnp.zeros((NP, W128), jnp.float32)

    d0, d1 = _deg_call(dst32, ones_g, zeros_n)

    h0, h1 = _mm1(x, W1, b1, d0, d1)
    a0, a1 = _pass_feat_call(h0, h1, src16, dst16)

    hws2 = _mm2(_mm_cat_body, a0, a1, W2, b2, d0, d1)
    a0, a1 = _pass_edge_call(hws2, zeros_n, src32, dst32)

    Wml = jnp.concatenate([Wmu, Wls], axis=1)
    bml = jnp.concatenate([bmu, bls], axis=0)
    hws3 = _mm2(_mm_add_body, a0, a1, Wml, bml, d0, d1)
    a0, a1 = _pass_edge_call(hws3, zeros_n, src32, dst32)

    eps = jax.random.normal(jax.random.key(42), (N, OUT), dtype=jnp.float32)
    return _zfin(a0, a1, d0, d1, eps)


# speedup vs baseline: 16.3551x; 1.0553x over previous
import functools

import jax
import jax.numpy as jnp
from jax import lax
from jax.experimental import pallas as pl
from jax.experimental.pallas import tpu as pltpu
from jax.experimental.pallas import tpu_sc as plsc

N = 10000
E = 160000
IN_DIMS = 256
HID1 = 256
HID2 = 128
OUT = 64

NC = 2
NS = 16
G = 125
NCH = E // (NS * G)
NCH_D = E // (NC * NS * G)
NP = 10240
RPT = NP // NS
W128 = 128

_MESH = plsc.VectorSubcoreMesh(core_axis_name="c", subcore_axis_name="s")


NBUF = 2


def _pipelined_scatter_gather(table, out, src_v, dst_v, bufs, ssems,
                              acc, ngroups, r0, init_ref, init_sem):
    init_cp = pltpu.async_copy(
        init_ref.at[pl.ds(r0, RPT)], acc.at[pl.ds(r0, RPT)], init_sem)
    init_cp.wait()
    plsc.subcore_barrier()

    def group(i, carry):
        scps = []
        for b in range(NBUF):
            j = i * NBUF + b
            pltpu.sync_copy(table.at[src_v.at[j]], bufs[b])
            scps.append(
                pltpu.async_copy(bufs[b], acc.at[dst_v.at[j]], ssems[b],
                                 add=True))
        for b in range(NBUF):
            scps[b].wait()
        return carry

    lax.fori_loop(0, ngroups, group, 0)
    plsc.subcore_barrier()
    pltpu.sync_copy(acc.at[pl.ds(r0, RPT)], out.at[pl.ds(r0, RPT)])


def _deg_body(dst_hbm, ones_hbm, zeros_hbm, d0_hbm, d1_hbm, dst_v, ones_v,
              acc, init_sem, *ssems):
    c = lax.axis_index("c")
    s = lax.axis_index("s")
    w = c * NS + s
    pltpu.sync_copy(dst_hbm.at[w], dst_v)
    pltpu.sync_copy(ones_hbm, ones_v)
    r0 = s * RPT
    init_cp = pltpu.async_copy(
        zeros_hbm.at[pl.ds(r0, RPT)], acc.at[pl.ds(r0, RPT)], init_sem)
    init_cp.wait()
    plsc.subcore_barrier()

    def group(i, carry):
        scps = []
        for b in range(NBUF):
            j = i * NBUF + b
            scps.append(
                pltpu.async_copy(ones_v, acc.at[dst_v.at[j]], ssems[b],
                                 add=True))
        for b in range(NBUF):
            scps[b].wait()
        return carry

    lax.fori_loop(0, NCH_D // NBUF, group, 0)
    plsc.subcore_barrier()

    def wout(out):
        pltpu.sync_copy(acc.at[pl.ds(r0, RPT)], out.at[pl.ds(r0, RPT)])

    pl.when(c == 0)(lambda: wout(d0_hbm))
    pl.when(c == 1)(lambda: wout(d1_hbm))


_deg_call = pl.kernel(
    _deg_body,
    out_type=[jax.ShapeDtypeStruct((NP, W128), jnp.float32)] * 2,
    mesh=_MESH,
    scratch_types=[
        pltpu.VMEM((NCH_D, G), jnp.int32),
        pltpu.VMEM((G, W128), jnp.float32),
        pltpu.VMEM_SHARED((NP, W128), jnp.float32),
        pltpu.SemaphoreType.DMA,
    ] + [pltpu.SemaphoreType.DMA] * NBUF,
)


def _pass_edge_body(hws, zeros_hbm, src_hbm, dst_hbm, out0, out1,
                    src_v, dst_v, b0, b1, acc, init_sem, *ssems):
    c = lax.axis_index("c")
    s = lax.axis_index("s")
    w = c * NS + s
    pltpu.sync_copy(src_hbm.at[w], src_v)
    pltpu.sync_copy(dst_hbm.at[w], dst_v)
    r0 = s * RPT
    bufs = (b0, b1)

    def work(init_ref, out):
        _pipelined_scatter_gather(hws, out, src_v, dst_v, bufs,
                                  ssems, acc, NCH_D // NBUF, r0,
                                  init_ref, init_sem)

    pl.when(c == 0)(lambda: work(hws, out0))
    pl.when(c == 1)(lambda: work(zeros_hbm, out1))


_pass_edge_call = pl.kernel(
    _pass_edge_body,
    out_type=[jax.ShapeDtypeStruct((NP, W128), jnp.float32)] * 2,
    mesh=_MESH,
    scratch_types=[
        pltpu.VMEM((NCH_D, G), jnp.int32),
        pltpu.VMEM((NCH_D, G), jnp.int32),
    ] + [pltpu.VMEM((G, W128), jnp.float32)] * NBUF + [
        pltpu.VMEM_SHARED((NP, W128), jnp.float32),
        pltpu.SemaphoreType.DMA,
    ] + [pltpu.SemaphoreType.DMA] * NBUF,
)


TM = 1000


def _dinv_of(d0, d1):
    deg = d0[:, 0:1] + d1[:, 0:1] + 1.0
    return lax.rsqrt(deg)


def _mm1_body(x_ref, w_ref, b_ref, d0_ref, d1_ref, o0_ref, o1_ref):
    dinv = _dinv_of(d0_ref[...], d1_ref[...])
    hw = jnp.dot(x_ref[...], w_ref[...], preferred_element_type=jnp.float32)
    hws = (hw + b_ref[...]) * dinv
    half = hws.shape[1] // 2
    o0_ref[...] = hws[:, :half]
    o1_ref[...] = hws[:, half:]


def _mm_cat_body(a0_ref, a1_ref, a2_ref, a3_ref, w_ref, b_ref, d0_ref,
                 d1_ref, o_ref):
    dinv = _dinv_of(d0_ref[...], d1_ref[...])
    h = jnp.concatenate([a0_ref[...] + a1_ref[...],
                         a2_ref[...] + a3_ref[...]], axis=1) * dinv
    h = jnp.maximum(h, 0.0)
    hw = jnp.dot(h, w_ref[...], preferred_element_type=jnp.float32)
    o_ref[...] = (hw + b_ref[...]) * dinv


def _mm_add_body(a0_ref, a1_ref, w_ref, b_ref, d0_ref, d1_ref, o_ref):
    dinv = _dinv_of(d0_ref[...], d1_ref[...])
    h = (a0_ref[...] + a1_ref[...]) * dinv
    h = jnp.maximum(h, 0.0)
    hw = jnp.dot(h, w_ref[...], preferred_element_type=jnp.float32)
    o_ref[...] = (hw + b_ref[...]) * dinv


def _z_body(a0_ref, a1_ref, d0_ref, d1_ref, eps_ref, z_ref):
    dinv = _dinv_of(d0_ref[...], d1_ref[...])
    ml = (a0_ref[...] + a1_ref[...]) * dinv
    mu = ml[:, :OUT]
    logstd = jnp.minimum(ml[:, OUT:], 10.0)
    z_ref[...] = mu + eps_ref[...] * jnp.exp(logstd)


def _row_spec(cols):
    return pl.BlockSpec((TM, cols), lambda i: (i, 0))


def _full_spec(shape):
    return pl.BlockSpec(shape, lambda i: (0,) * len(shape))


def _mm1(x, W, b, d0, d1):
    dout = W.shape[1]
    return pl.pallas_call(
        _mm1_body,
        grid=(N // TM,),
        in_specs=[
            _row_spec(x.shape[1]),
            _full_spec(W.shape),
            _full_spec((1, dout)),
            _row_spec(W128),
            _row_spec(W128),
        ],
        out_specs=[_row_spec(dout // 2)] * 2,
        out_shape=[jax.ShapeDtypeStruct((NP, dout // 2), jnp.float32)] * 2,
    )(x, W, b.reshape(1, -1), d0, d1)


def _mm2(body, aparts, W, b, d0, d1):
    dout = W.shape[1]
    return pl.pallas_call(
        body,
        grid=(N // TM,),
        in_specs=[_row_spec(a.shape[1]) for a in aparts] + [
            _full_spec(W.shape),
            _full_spec((1, dout)),
            _row_spec(W128),
            _row_spec(W128),
        ],
        out_specs=_row_spec(dout),
        out_shape=jax.ShapeDtypeStruct((NP, dout), jnp.float32),
    )(*aparts, W, b.reshape(1, -1), d0, d1)


def _zfin(a0, a1, d0, d1, eps):
    return pl.pallas_call(
        _z_body,
        grid=(N // TM,),
        in_specs=[
            _row_spec(W128),
            _row_spec(W128),
            _row_spec(W128),
            _row_spec(W128),
            _row_spec(OUT),
        ],
        out_specs=_row_spec(OUT),
        out_shape=jax.ShapeDtypeStruct((N, OUT), jnp.float32),
    )(a0, a1, d0, d1, eps)


def kernel(x, edge_index, W1, b1, W2, b2, Wmu, bmu, Wls, bls):
    src = edge_index[0].astype(jnp.int32)
    dst = edge_index[1].astype(jnp.int32)
    src32 = src.reshape(NC * NS, NCH_D, G)
    dst32 = dst.reshape(NC * NS, NCH_D, G)

    ones_g = jnp.ones((G, W128), jnp.float32)
    zeros_n = jnp.zeros((NP, W128), jnp.float32)

    d0, d1 = _deg_call(dst32, ones_g, zeros_n)

    h0, h1 = _mm1(x, W1, b1, d0, d1)
    p0a, p0b = _pass_edge_call(h0, zeros_n, src32, dst32)
    p1a, p1b = _pass_edge_call(h1, zeros_n, src32, dst32)

    hws2 = _mm2(_mm_cat_body, (p0a, p0b, p1a, p1b), W2, b2, d0, d1)
    a0, a1 = _pass_edge_call(hws2, zeros_n, src32, dst32)

    Wml = jnp.concatenate([Wmu, Wls], axis=1)
    bml = jnp.concatenate([bmu, bls], axis=0)
    hws3 = _mm2(_mm_add_body, (a0, a1), Wml, bml, d0, d1)
    a0, a1 = _pass_edge_call(hws3, zeros_n, src32, dst32)

    eps = jax.random.normal(jax.random.key(42), (N, OUT), dtype=jnp.float32)
    return _zfin(a0, a1, d0, d1, eps)


# speedup vs baseline: 16.6441x; 1.0177x over previous
import functools

import jax
import jax.numpy as jnp
from jax import lax
from jax.experimental import pallas as pl
from jax.experimental.pallas import tpu as pltpu
from jax.experimental.pallas import tpu_sc as plsc

N = 10000
E = 160000
IN_DIMS = 256
HID1 = 256
HID2 = 128
OUT = 64

NC = 2
NS = 16
G = 125
NCH = E // (NS * G)
NCH_D = E // (NC * NS * G)
NP = 10240
RPT = NP // NS
W128 = 128

_MESH = plsc.VectorSubcoreMesh(core_axis_name="c", subcore_axis_name="s")


NBUF = 2


def _pipelined_scatter_gather(table, out, src_v, dst_v, bufs, gsems, ssems,
                              acc, ngroups, r0, init_ref, init_sem):
    init_cp = pltpu.async_copy(
        init_ref.at[pl.ds(r0, RPT)], acc.at[pl.ds(r0, RPT)], init_sem)
    init_cp.wait()
    plsc.subcore_barrier()

    def group(i, carry):
        gcps = []
        for b in range(NBUF):
            j = i * NBUF + b
            gcps.append(
                pltpu.async_copy(table.at[src_v.at[j]], bufs[b], gsems[b]))
        scps = []
        for b in range(NBUF):
            j = i * NBUF + b
            gcps[b].wait()
            scps.append(
                pltpu.async_copy(bufs[b], acc.at[dst_v.at[j]], ssems[b],
                                 add=True))
        for b in range(NBUF):
            scps[b].wait()
        return carry

    lax.fori_loop(0, ngroups, group, 0)
    plsc.subcore_barrier()
    pltpu.sync_copy(acc.at[pl.ds(r0, RPT)], out.at[pl.ds(r0, RPT)])


def _deg_body(dst_hbm, ones_hbm, zeros_hbm, d0_hbm, d1_hbm, dst_v, ones_v,
              acc, init_sem, *ssems):
    c = lax.axis_index("c")
    s = lax.axis_index("s")
    w = c * NS + s
    pltpu.sync_copy(dst_hbm.at[w], dst_v)
    pltpu.sync_copy(ones_hbm, ones_v)
    r0 = s * RPT
    init_cp = pltpu.async_copy(
        zeros_hbm.at[pl.ds(r0, RPT)], acc.at[pl.ds(r0, RPT)], init_sem)
    init_cp.wait()
    plsc.subcore_barrier()

    def group(i, carry):
        scps = []
        for b in range(NBUF):
            j = i * NBUF + b
            scps.append(
                pltpu.async_copy(ones_v, acc.at[dst_v.at[j]], ssems[b],
                                 add=True))
        for b in range(NBUF):
            scps[b].wait()
        return carry

    lax.fori_loop(0, NCH_D // NBUF, group, 0)
    plsc.subcore_barrier()

    def wout(out):
        pltpu.sync_copy(acc.at[pl.ds(r0, RPT)], out.at[pl.ds(r0, RPT)])

    pl.when(c == 0)(lambda: wout(d0_hbm))
    pl.when(c == 1)(lambda: wout(d1_hbm))


_deg_call = pl.kernel(
    _deg_body,
    out_type=[jax.ShapeDtypeStruct((NP, W128), jnp.float32)] * 2,
    mesh=_MESH,
    scratch_types=[
        pltpu.VMEM((NCH_D, G), jnp.int32),
        pltpu.VMEM((G, W128), jnp.float32),
        pltpu.VMEM_SHARED((NP, W128), jnp.float32),
        pltpu.SemaphoreType.DMA,
    ] + [pltpu.SemaphoreType.DMA] * NBUF,
)


def _pass_edge_body(hws, zeros_hbm, src_hbm, dst_hbm, out0, out1,
                    src_v, dst_v, b0, b1, acc, init_sem, *sems):
    c = lax.axis_index("c")
    s = lax.axis_index("s")
    w = c * NS + s
    pltpu.sync_copy(src_hbm.at[w], src_v)
    pltpu.sync_copy(dst_hbm.at[w], dst_v)
    r0 = s * RPT
    bufs = (b0, b1)
    gsems, ssems = sems[:NBUF], sems[NBUF:]

    def work(init_ref, out):
        _pipelined_scatter_gather(hws, out, src_v, dst_v, bufs, gsems,
                                  ssems, acc, NCH_D // NBUF, r0,
                                  init_ref, init_sem)

    pl.when(c == 0)(lambda: work(hws, out0))
    pl.when(c == 1)(lambda: work(zeros_hbm, out1))


_pass_edge_call = pl.kernel(
    _pass_edge_body,
    out_type=[jax.ShapeDtypeStruct((NP, W128), jnp.float32)] * 2,
    mesh=_MESH,
    scratch_types=[
        pltpu.VMEM((NCH_D, G), jnp.int32),
        pltpu.VMEM((NCH_D, G), jnp.int32),
    ] + [pltpu.VMEM((G, W128), jnp.float32)] * NBUF + [
        pltpu.VMEM_SHARED((NP, W128), jnp.float32),
        pltpu.SemaphoreType.DMA,
    ] + [pltpu.SemaphoreType.DMA] * (2 * NBUF),
)


TM = 1000


def _dinv_of(d0, d1):
    deg = d0[:, 0:1] + d1[:, 0:1] + 1.0
    return lax.rsqrt(deg)


def _mm1_body(x_ref, w_ref, b_ref, d0_ref, d1_ref, o0_ref, o1_ref):
    dinv = _dinv_of(d0_ref[...], d1_ref[...])
    hw = jnp.dot(x_ref[...], w_ref[...], preferred_element_type=jnp.float32)
    hws = (hw + b_ref[...]) * dinv
    half = hws.shape[1] // 2
    o0_ref[...] = hws[:, :half]
    o1_ref[...] = hws[:, half:]


def _mm_cat_body(a0_ref, a1_ref, a2_ref, a3_ref, w_ref, b_ref, d0_ref,
                 d1_ref, o_ref):
    dinv = _dinv_of(d0_ref[...], d1_ref[...])
    h = jnp.concatenate([a0_ref[...] + a1_ref[...],
                         a2_ref[...] + a3_ref[...]], axis=1) * dinv
    h = jnp.maximum(h, 0.0)
    hw = jnp.dot(h, w_ref[...], preferred_element_type=jnp.float32)
    o_ref[...] = (hw + b_ref[...]) * dinv


def _mm_add_body(a0_ref, a1_ref, w_ref, b_ref, d0_ref, d1_ref, o_ref):
    dinv = _dinv_of(d0_ref[...], d1_ref[...])
    h = (a0_ref[...] + a1_ref[...]) * dinv
    h = jnp.maximum(h, 0.0)
    hw = jnp.dot(h, w_ref[...], preferred_element_type=jnp.float32)
    o_ref[...] = (hw + b_ref[...]) * dinv


def _z_body(a0_ref, a1_ref, d0_ref, d1_ref, eps_ref, z_ref):
    dinv = _dinv_of(d0_ref[...], d1_ref[...])
    ml = (a0_ref[...] + a1_ref[...]) * dinv
    mu = ml[:, :OUT]
    logstd = jnp.minimum(ml[:, OUT:], 10.0)
    z_ref[...] = mu + eps_ref[...] * jnp.exp(logstd)


def _row_spec(cols):
    return pl.BlockSpec((TM, cols), lambda i: (i, 0))


def _full_spec(shape):
    return pl.BlockSpec(shape, lambda i: (0,) * len(shape))


def _mm1(x, W, b, d0, d1):
    dout = W.shape[1]
    return pl.pallas_call(
        _mm1_body,
        grid=(N // TM,),
        in_specs=[
            _row_spec(x.shape[1]),
            _full_spec(W.shape),
            _full_spec((1, dout)),
            _row_spec(W128),
            _row_spec(W128),
        ],
        out_specs=[_row_spec(dout // 2)] * 2,
        out_shape=[jax.ShapeDtypeStruct((NP, dout // 2), jnp.float32)] * 2,
    )(x, W, b.reshape(1, -1), d0, d1)


def _mm2(body, aparts, W, b, d0, d1):
    dout = W.shape[1]
    return pl.pallas_call(
        body,
        grid=(N // TM,),
        in_specs=[_row_spec(a.shape[1]) for a in aparts] + [
            _full_spec(W.shape),
            _full_spec((1, dout)),
            _row_spec(W128),
            _row_spec(W128),
        ],
        out_specs=_row_spec(dout),
        out_shape=jax.ShapeDtypeStruct((NP, dout), jnp.float32),
    )(*aparts, W, b.reshape(1, -1), d0, d1)


def _zfin(a0, a1, d0, d1, eps):
    return pl.pallas_call(
        _z_body,
        grid=(N // TM,),
        in_specs=[
            _row_spec(W128),
            _row_spec(W128),
            _row_spec(W128),
            _row_spec(W128),
            _row_spec(OUT),
        ],
        out_specs=_row_spec(OUT),
        out_shape=jax.ShapeDtypeStruct((N, OUT), jnp.float32),
    )(a0, a1, d0, d1, eps)


def kernel(x, edge_index, W1, b1, W2, b2, Wmu, bmu, Wls, bls):
    src = edge_index[0].astype(jnp.int32)
    dst = edge_index[1].astype(jnp.int32)
    src32 = src.reshape(NC * NS, NCH_D, G)
    dst32 = dst.reshape(NC * NS, NCH_D, G)

    ones_g = jnp.ones((G, W128), jnp.float32)
    zeros_n = jnp.zeros((NP, W128), jnp.float32)

    d0, d1 = _deg_call(dst32, ones_g, zeros_n)

    h0, h1 = _mm1(x, W1, b1, d0, d1)
    p0a, p0b = _pass_edge_call(h0, zeros_n, src32, dst32)
    p1a, p1b = _pass_edge_call(h1, zeros_n, src32, dst32)

    hws2 = _mm2(_mm_cat_body, (p0a, p0b, p1a, p1b), W2, b2, d0, d1)
    a0, a1 = _pass_edge_call(hws2, zeros_n, src32, dst32)

    Wml = jnp.concatenate([Wmu, Wls], axis=1)
    bml = jnp.concatenate([bmu, bls], axis=0)
    hws3 = _mm2(_mm_add_body, (a0, a1), Wml, bml, d0, d1)
    a0, a1 = _pass_edge_call(hws3, zeros_n, src32, dst32)

    eps = jax.random.normal(jax.random.key(42), (N, OUT), dtype=jnp.float32)
    return _zfin(a0, a1, d0, d1, eps)


# speedup vs baseline: 16.7880x; 1.0086x over previous
import functools

import jax
import jax.numpy as jnp
from jax import lax
from jax.experimental import pallas as pl
from jax.experimental.pallas import tpu as pltpu
from jax.experimental.pallas import tpu_sc as plsc

N = 10000
E = 160000
IN_DIMS = 256
HID1 = 256
HID2 = 128
OUT = 64

NC = 2
NS = 16
G = 125
NCH = E // (NS * G)
NCH_D = E // (NC * NS * G)
NP = 10240
RPT = NP // NS
W128 = 128

_MESH = plsc.VectorSubcoreMesh(core_axis_name="c", subcore_axis_name="s")


NBUF = 2


def _pipelined_scatter_gather(table, out, src_v, dst_v, bufs, gsems, ssems,
                              acc, ngroups, r0, init_ref, init_sem):
    init_cp = pltpu.async_copy(
        init_ref.at[pl.ds(r0, RPT)], acc.at[pl.ds(r0, RPT)], init_sem)
    init_cp.wait()
    plsc.subcore_barrier()

    def group(i, carry):
        gcps = []
        for b in range(NBUF):
            j = i * NBUF + b
            gcps.append(
                pltpu.async_copy(table.at[src_v.at[j]], bufs[b], gsems[b]))
        scps = []
        for b in range(NBUF):
            j = i * NBUF + b
            gcps[b].wait()
            scps.append(
                pltpu.async_copy(bufs[b], acc.at[dst_v.at[j]], ssems[b],
                                 add=True))
        for b in range(NBUF):
            scps[b].wait()
        return carry

    lax.fori_loop(0, ngroups, group, 0)
    plsc.subcore_barrier()
    pltpu.sync_copy(acc.at[pl.ds(r0, RPT)], out.at[pl.ds(r0, RPT)])


def _deg_body(dst_hbm, ones_hbm, zeros_hbm, d0_hbm, d1_hbm, dst_v, ones_v,
              acc, init_sem, *ssems):
    c = lax.axis_index("c")
    s = lax.axis_index("s")
    w = c * NS + s
    pltpu.sync_copy(dst_hbm.at[w], dst_v)
    pltpu.sync_copy(ones_hbm, ones_v)
    r0 = s * RPT
    init_cp = pltpu.async_copy(
        zeros_hbm.at[pl.ds(r0, RPT)], acc.at[pl.ds(r0, RPT)], init_sem)
    init_cp.wait()
    plsc.subcore_barrier()

    def group(i, carry):
        scps = []
        for b in range(NBUF):
            j = i * NBUF + b
            scps.append(
                pltpu.async_copy(ones_v, acc.at[dst_v.at[j]], ssems[b],
                                 add=True))
        for b in range(NBUF):
            scps[b].wait()
        return carry

    lax.fori_loop(0, NCH_D // NBUF, group, 0)
    plsc.subcore_barrier()

    def wout(out):
        pltpu.sync_copy(acc.at[pl.ds(r0, RPT)], out.at[pl.ds(r0, RPT)])

    pl.when(c == 0)(lambda: wout(d0_hbm))
    pl.when(c == 1)(lambda: wout(d1_hbm))


_deg_call = pl.kernel(
    _deg_body,
    out_type=[jax.ShapeDtypeStruct((NP, W128), jnp.float32)] * 2,
    mesh=_MESH,
    scratch_types=[
        pltpu.VMEM((NCH_D, G), jnp.int32),
        pltpu.VMEM((G, W128), jnp.float32),
        pltpu.VMEM_SHARED((NP, W128), jnp.float32),
        pltpu.SemaphoreType.DMA,
    ] + [pltpu.SemaphoreType.DMA] * NBUF,
)


def _pass_edge_body(hws, zeros_hbm, src_hbm, dst_hbm, out0, out1,
                    src_v, dst_v, b0, b1, acc, init_sem, *sems):
    c = lax.axis_index("c")
    s = lax.axis_index("s")
    w = c * NS + s
    pltpu.sync_copy(src_hbm.at[w], src_v)
    pltpu.sync_copy(dst_hbm.at[w], dst_v)
    r0 = s * RPT
    bufs = (b0, b1)
    gsems, ssems = sems[:NBUF], sems[NBUF:]

    def work(init_ref, out):
        _pipelined_scatter_gather(hws, out, src_v, dst_v, bufs, gsems,
                                  ssems, acc, NCH_D // NBUF, r0,
                                  init_ref, init_sem)

    pl.when(c == 0)(lambda: work(hws, out0))
    pl.when(c == 1)(lambda: work(zeros_hbm, out1))


_pass_edge_call = pl.kernel(
    _pass_edge_body,
    out_type=[jax.ShapeDtypeStruct((NP, W128), jnp.float32)] * 2,
    mesh=_MESH,
    scratch_types=[
        pltpu.VMEM((NCH_D, G), jnp.int32),
        pltpu.VMEM((NCH_D, G), jnp.int32),
    ] + [pltpu.VMEM((G, W128), jnp.float32)] * NBUF + [
        pltpu.VMEM_SHARED((NP, W128), jnp.float32),
        pltpu.SemaphoreType.DMA,
    ] + [pltpu.SemaphoreType.DMA] * (2 * NBUF),
)


def _pass_edge2_body(hA, hB, zeros_hbm, src_hbm, dst_hbm,
                     outA0, outA1, outB0, outB1,
                     src_v, dst_v, b0, b1, acc, init_sem, *sems):
    c = lax.axis_index("c")
    s = lax.axis_index("s")
    w = c * NS + s
    pltpu.sync_copy(src_hbm.at[w], src_v)
    pltpu.sync_copy(dst_hbm.at[w], dst_v)
    r0 = s * RPT
    bufs = (b0, b1)
    gsems, ssems = sems[:NBUF], sems[NBUF:]

    def work(table, init_ref, out):
        _pipelined_scatter_gather(table, out, src_v, dst_v, bufs, gsems,
                                  ssems, acc, NCH_D // NBUF, r0,
                                  init_ref, init_sem)

    pl.when(c == 0)(lambda: work(hA, hA, outA0))
    pl.when(c == 1)(lambda: work(hA, zeros_hbm, outA1))
    plsc.subcore_barrier()
    pl.when(c == 0)(lambda: work(hB, hB, outB0))
    pl.when(c == 1)(lambda: work(hB, zeros_hbm, outB1))


_pass_edge2_call = pl.kernel(
    _pass_edge2_body,
    out_type=[jax.ShapeDtypeStruct((NP, W128), jnp.float32)] * 4,
    mesh=_MESH,
    scratch_types=[
        pltpu.VMEM((NCH_D, G), jnp.int32),
        pltpu.VMEM((NCH_D, G), jnp.int32),
    ] + [pltpu.VMEM((G, W128), jnp.float32)] * NBUF + [
        pltpu.VMEM_SHARED((NP, W128), jnp.float32),
        pltpu.SemaphoreType.DMA,
    ] + [pltpu.SemaphoreType.DMA] * (2 * NBUF),
)


TM = 1000


def _dinv_of(d0, d1):
    deg = d0[:, 0:1] + d1[:, 0:1] + 1.0
    return lax.rsqrt(deg)


def _mm1_body(x_ref, w_ref, b_ref, d0_ref, d1_ref, o0_ref, o1_ref):
    dinv = _dinv_of(d0_ref[...], d1_ref[...])
    hw = jnp.dot(x_ref[...], w_ref[...], preferred_element_type=jnp.float32)
    hws = (hw + b_ref[...]) * dinv
    half = hws.shape[1] // 2
    o0_ref[...] = hws[:, :half]
    o1_ref[...] = hws[:, half:]


def _mm_cat_body(a0_ref, a1_ref, a2_ref, a3_ref, w_ref, b_ref, d0_ref,
                 d1_ref, o_ref):
    dinv = _dinv_of(d0_ref[...], d1_ref[...])
    h = jnp.concatenate([a0_ref[...] + a1_ref[...],
                         a2_ref[...] + a3_ref[...]], axis=1) * dinv
    h = jnp.maximum(h, 0.0)
    hw = jnp.dot(h, w_ref[...], preferred_element_type=jnp.float32)
    o_ref[...] = (hw + b_ref[...]) * dinv


def _mm_add_body(a0_ref, a1_ref, w_ref, b_ref, d0_ref, d1_ref, o_ref):
    dinv = _dinv_of(d0_ref[...], d1_ref[...])
    h = (a0_ref[...] + a1_ref[...]) * dinv
    h = jnp.maximum(h, 0.0)
    hw = jnp.dot(h, w_ref[...], preferred_element_type=jnp.float32)
    o_ref[...] = (hw + b_ref[...]) * dinv


def _z_body(a0_ref, a1_ref, d0_ref, d1_ref, eps_ref, z_ref):
    dinv = _dinv_of(d0_ref[...], d1_ref[...])
    ml = (a0_ref[...] + a1_ref[...]) * dinv
    mu = ml[:, :OUT]
    logstd = jnp.minimum(ml[:, OUT:], 10.0)
    z_ref[...] = mu + eps_ref[...] * jnp.exp(logstd)


def _row_spec(cols):
    return pl.BlockSpec((TM, cols), lambda i: (i, 0))


def _full_spec(shape):
    return pl.BlockSpec(shape, lambda i: (0,) * len(shape))


def _mm1(x, W, b, d0, d1):
    dout = W.shape[1]
    return pl.pallas_call(
        _mm1_body,
        grid=(N // TM,),
        in_specs=[
            _row_spec(x.shape[1]),
            _full_spec(W.shape),
            _full_spec((1, dout)),
            _row_spec(W128),
            _row_spec(W128),
        ],
        out_specs=[_row_spec(dout // 2)] * 2,
        out_shape=[jax.ShapeDtypeStruct((NP, dout // 2), jnp.float32)] * 2,
    )(x, W, b.reshape(1, -1), d0, d1)


def _mm2(body, aparts, W, b, d0, d1):
    dout = W.shape[1]
    return pl.pallas_call(
        body,
        grid=(N // TM,),
        in_specs=[_row_spec(a.shape[1]) for a in aparts] + [
            _full_spec(W.shape),
            _full_spec((1, dout)),
            _row_spec(W128),
            _row_spec(W128),
        ],
        out_specs=_row_spec(dout),
        out_shape=jax.ShapeDtypeStruct((NP, dout), jnp.float32),
    )(*aparts, W, b.reshape(1, -1), d0, d1)


def _zfin(a0, a1, d0, d1, eps):
    return pl.pallas_call(
        _z_body,
        grid=(N // TM,),
        in_specs=[
            _row_spec(W128),
            _row_spec(W128),
            _row_spec(W128),
            _row_spec(W128),
            _row_spec(OUT),
        ],
        out_specs=_row_spec(OUT),
        out_shape=jax.ShapeDtypeStruct((N, OUT), jnp.float32),
    )(a0, a1, d0, d1, eps)


def kernel(x, edge_index, W1, b1, W2, b2, Wmu, bmu, Wls, bls):
    src = edge_index[0].astype(jnp.int32)
    dst = edge_index[1].astype(jnp.int32)
    src32 = src.reshape(NC * NS, NCH_D, G)
    dst32 = dst.reshape(NC * NS, NCH_D, G)

    ones_g = jnp.ones((G, W128), jnp.float32)
    zeros_n = jnp.zeros((NP, W128), jnp.float32)

    d0, d1 = _deg_call(dst32, ones_g, zeros_n)

    h0, h1 = _mm1(x, W1, b1, d0, d1)
    p0a, p0b, p1a, p1b = _pass_edge2_call(h0, h1, zeros_n, src32, dst32)

    hws2 = _mm2(_mm_cat_body, (p0a, p0b, p1a, p1b), W2, b2, d0, d1)
    a0, a1 = _pass_edge_call(hws2, zeros_n, src32, dst32)

    Wml = jnp.concatenate([Wmu, Wls], axis=1)
    bml = jnp.concatenate([bmu, bls], axis=0)
    hws3 = _mm2(_mm_add_body, (a0, a1), Wml, bml, d0, d1)
    a0, a1 = _pass_edge_call(hws3, zeros_n, src32, dst32)

    eps = jax.random.normal(jax.random.key(42), (N, OUT), dtype=jnp.float32)
    return _zfin(a0, a1, d0, d1, eps)


# speedup vs baseline: 16.8707x; 1.0049x over previous
import functools

import jax
import jax.numpy as jnp
from jax import lax
from jax.experimental import pallas as pl
from jax.experimental.pallas import tpu as pltpu
from jax.experimental.pallas import tpu_sc as plsc

N = 10000
E = 160000
IN_DIMS = 256
HID1 = 256
HID2 = 128
OUT = 64

NC = 2
NS = 16
G = 125
NCH = E // (NS * G)
NCH_D = E // (NC * NS * G)
NP = 10240
RPT = NP // NS
W128 = 128

_MESH = plsc.VectorSubcoreMesh(core_axis_name="c", subcore_axis_name="s")


NBUF = 2


def _pipelined_scatter_gather(table, out, src_v, dst_v, bufs, gsems, ssems,
                              acc, ngroups, r0, init_ref, init_sem):
    init_cp = pltpu.async_copy(
        init_ref.at[pl.ds(r0, RPT)], acc.at[pl.ds(r0, RPT)], init_sem)
    init_cp.wait()
    plsc.subcore_barrier()

    def group(i, carry):
        gcps = []
        for b in range(NBUF):
            j = i * NBUF + b
            gcps.append(
                pltpu.async_copy(table.at[src_v.at[j]], bufs[b], gsems[b]))
        scps = []
        for b in range(NBUF):
            j = i * NBUF + b
            gcps[b].wait()
            scps.append(
                pltpu.async_copy(bufs[b], acc.at[dst_v.at[j]], ssems[b],
                                 add=True))
        for b in range(NBUF):
            scps[b].wait()
        return carry

    lax.fori_loop(0, ngroups, group, 0)
    plsc.subcore_barrier()
    pltpu.sync_copy(acc.at[pl.ds(r0, RPT)], out.at[pl.ds(r0, RPT)])


def _deg_body(dst_hbm, ones_hbm, zeros_hbm, d0_hbm, d1_hbm, dst_v, ones_v,
              acc, init_sem, *ssems):
    c = lax.axis_index("c")
    s = lax.axis_index("s")
    w = c * NS + s
    pltpu.sync_copy(dst_hbm.at[w], dst_v)
    pltpu.sync_copy(ones_hbm, ones_v)
    r0 = s * RPT
    init_cp = pltpu.async_copy(
        zeros_hbm.at[pl.ds(r0, RPT)], acc.at[pl.ds(r0, RPT)], init_sem)
    init_cp.wait()
    plsc.subcore_barrier()

    def group(i, carry):
        scps = []
        for b in range(NBUF):
            j = i * NBUF + b
            scps.append(
                pltpu.async_copy(ones_v, acc.at[dst_v.at[j]], ssems[b],
                                 add=True))
        for b in range(NBUF):
            scps[b].wait()
        return carry

    lax.fori_loop(0, NCH_D // NBUF, group, 0)
    plsc.subcore_barrier()

    def wout(out):
        pltpu.sync_copy(acc.at[pl.ds(r0, RPT)], out.at[pl.ds(r0, RPT)])

    pl.when(c == 0)(lambda: wout(d0_hbm))
    pl.when(c == 1)(lambda: wout(d1_hbm))


_deg_call = pl.kernel(
    _deg_body,
    out_type=[jax.ShapeDtypeStruct((NP, W128), jnp.float32)] * 2,
    mesh=_MESH,
    scratch_types=[
        pltpu.VMEM((NCH_D, G), jnp.int32),
        pltpu.VMEM((G, W128), jnp.float32),
        pltpu.VMEM_SHARED((NP, W128), jnp.float32),
        pltpu.SemaphoreType.DMA,
    ] + [pltpu.SemaphoreType.DMA] * NBUF,
)


def _pass_edge_body(hws, zeros_hbm, src_hbm, dst_hbm, out0, out1,
                    src_v, dst_v, b0, b1, acc, init_sem, *sems):
    c = lax.axis_index("c")
    s = lax.axis_index("s")
    w = c * NS + s
    pltpu.sync_copy(src_hbm.at[w], src_v)
    pltpu.sync_copy(dst_hbm.at[w], dst_v)
    r0 = s * RPT
    bufs = (b0, b1)
    gsems, ssems = sems[:NBUF], sems[NBUF:]

    def work(init_ref, out):
        _pipelined_scatter_gather(hws, out, src_v, dst_v, bufs, gsems,
                                  ssems, acc, NCH_D // NBUF, r0,
                                  init_ref, init_sem)

    pl.when(c == 0)(lambda: work(hws, out0))
    pl.when(c == 1)(lambda: work(zeros_hbm, out1))


_pass_edge_call = pl.kernel(
    _pass_edge_body,
    out_type=[jax.ShapeDtypeStruct((NP, W128), jnp.float32)] * 2,
    mesh=_MESH,
    scratch_types=[
        pltpu.VMEM((NCH_D, G), jnp.int32),
        pltpu.VMEM((NCH_D, G), jnp.int32),
    ] + [pltpu.VMEM((G, W128), jnp.float32)] * NBUF + [
        pltpu.VMEM_SHARED((NP, W128), jnp.float32),
        pltpu.SemaphoreType.DMA,
    ] + [pltpu.SemaphoreType.DMA] * (2 * NBUF),
)


def _pass_edge2_body(hA, hB, zeros_hbm, src_hbm, dst_hbm,
                     outA0, outA1, outB0, outB1,
                     src_v, dst_v, b0, b1, acc, init_sem, *sems):
    c = lax.axis_index("c")
    s = lax.axis_index("s")
    w = c * NS + s
    pltpu.sync_copy(src_hbm.at[w], src_v)
    pltpu.sync_copy(dst_hbm.at[w], dst_v)
    r0 = s * RPT
    bufs = (b0, b1)
    gsems, ssems = sems[:NBUF], sems[NBUF:]

    def work(table, init_ref, out):
        _pipelined_scatter_gather(table, out, src_v, dst_v, bufs, gsems,
                                  ssems, acc, NCH_D // NBUF, r0,
                                  init_ref, init_sem)

    pl.when(c == 0)(lambda: work(hA, hA, outA0))
    pl.when(c == 1)(lambda: work(hA, zeros_hbm, outA1))
    plsc.subcore_barrier()
    pl.when(c == 0)(lambda: work(hB, hB, outB0))
    pl.when(c == 1)(lambda: work(hB, zeros_hbm, outB1))


_pass_edge2_call = pl.kernel(
    _pass_edge2_body,
    out_type=[jax.ShapeDtypeStruct((NP, W128), jnp.float32)] * 4,
    mesh=_MESH,
    scratch_types=[
        pltpu.VMEM((NCH_D, G), jnp.int32),
        pltpu.VMEM((NCH_D, G), jnp.int32),
    ] + [pltpu.VMEM((G, W128), jnp.float32)] * NBUF + [
        pltpu.VMEM_SHARED((NP, W128), jnp.float32),
        pltpu.SemaphoreType.DMA,
    ] + [pltpu.SemaphoreType.DMA] * (2 * NBUF),
)


TM = 1000


def _dinv_of(d0, d1):
    deg = d0[:, 0:1] + d1[:, 0:1] + 1.0
    return lax.rsqrt(deg)


def _mm1_body(x_ref, w_ref, b_ref, d0_ref, d1_ref, o0_ref, o1_ref, dv_ref):
    dinv = _dinv_of(d0_ref[...], d1_ref[...])
    hw = jnp.dot(x_ref[...], w_ref[...], preferred_element_type=jnp.float32)
    hws = (hw + b_ref[...]) * dinv
    half = hws.shape[1] // 2
    o0_ref[...] = hws[:, :half]
    o1_ref[...] = hws[:, half:]
    dv_ref[...] = jnp.broadcast_to(dinv, (dinv.shape[0], 16))


def _mm_cat_body(a0_ref, a1_ref, a2_ref, a3_ref, w_ref, b_ref, dv_ref,
                 o_ref):
    dinv = dv_ref[:, 0:1]
    h = jnp.concatenate([a0_ref[...] + a1_ref[...],
                         a2_ref[...] + a3_ref[...]], axis=1) * dinv
    h = jnp.maximum(h, 0.0)
    hw = jnp.dot(h, w_ref[...], preferred_element_type=jnp.float32)
    o_ref[...] = (hw + b_ref[...]) * dinv


def _mm_add_body(a0_ref, a1_ref, w_ref, b_ref, dv_ref, o_ref):
    dinv = dv_ref[:, 0:1]
    h = (a0_ref[...] + a1_ref[...]) * dinv
    h = jnp.maximum(h, 0.0)
    hw = jnp.dot(h, w_ref[...], preferred_element_type=jnp.float32)
    o_ref[...] = (hw + b_ref[...]) * dinv


def _z_body(a0_ref, a1_ref, dv_ref, eps_ref, z_ref):
    dinv = dv_ref[:, 0:1]
    ml = (a0_ref[...] + a1_ref[...]) * dinv
    mu = ml[:, :OUT]
    logstd = jnp.minimum(ml[:, OUT:], 10.0)
    z_ref[...] = mu + eps_ref[...] * jnp.exp(logstd)


def _row_spec(cols):
    return pl.BlockSpec((TM, cols), lambda i: (i, 0))


def _full_spec(shape):
    return pl.BlockSpec(shape, lambda i: (0,) * len(shape))


def _mm1(x, W, b, d0, d1):
    dout = W.shape[1]
    return pl.pallas_call(
        _mm1_body,
        grid=(N // TM,),
        in_specs=[
            _row_spec(x.shape[1]),
            _full_spec(W.shape),
            _full_spec((1, dout)),
            _row_spec(W128),
            _row_spec(W128),
        ],
        out_specs=[_row_spec(dout // 2), _row_spec(dout // 2),
                   _row_spec(16)],
        out_shape=[jax.ShapeDtypeStruct((NP, dout // 2), jnp.float32)] * 2
        + [jax.ShapeDtypeStruct((NP, 16), jnp.float32)],
    )(x, W, b.reshape(1, -1), d0, d1)


def _mm2(body, aparts, W, b, dinv16):
    dout = W.shape[1]
    return pl.pallas_call(
        body,
        grid=(N // TM,),
        in_specs=[_row_spec(a.shape[1]) for a in aparts] + [
            _full_spec(W.shape),
            _full_spec((1, dout)),
            _row_spec(16),
        ],
        out_specs=_row_spec(dout),
        out_shape=jax.ShapeDtypeStruct((NP, dout), jnp.float32),
    )(*aparts, W, b.reshape(1, -1), dinv16)


def _zfin(a0, a1, dinv16, eps):
    return pl.pallas_call(
        _z_body,
        grid=(N // TM,),
        in_specs=[
            _row_spec(W128),
            _row_spec(W128),
            _row_spec(16),
            _row_spec(OUT),
        ],
        out_specs=_row_spec(OUT),
        out_shape=jax.ShapeDtypeStruct((N, OUT), jnp.float32),
    )(a0, a1, dinv16, eps)


def kernel(x, edge_index, W1, b1, W2, b2, Wmu, bmu, Wls, bls):
    src = edge_index[0].astype(jnp.int32)
    dst = edge_index[1].astype(jnp.int32)
    src32 = src.reshape(NC * NS, NCH_D, G)
    dst32 = dst.reshape(NC * NS, NCH_D, G)

    ones_g = jnp.ones((G, W128), jnp.float32)
    zeros_n = jnp.zeros((NP, W128), jnp.float32)

    d0, d1 = _deg_call(dst32, ones_g, zeros_n)

    h0, h1, dinv16 = _mm1(x, W1, b1, d0, d1)
    p0a, p0b, p1a, p1b = _pass_edge2_call(h0, h1, zeros_n, src32, dst32)

    hws2 = _mm2(_mm_cat_body, (p0a, p0b, p1a, p1b), W2, b2, dinv16)
    a0, a1 = _pass_edge_call(hws2, zeros_n, src32, dst32)

    Wml = jnp.concatenate([Wmu, Wls], axis=1)
    bml = jnp.concatenate([bmu, bls], axis=0)
    hws3 = _mm2(_mm_add_body, (a0, a1), Wml, bml, dinv16)
    a0, a1 = _pass_edge_call(hws3, zeros_n, src32, dst32)

    eps = jax.random.normal(jax.random.key(42), (N, OUT), dtype=jnp.float32)
    return _zfin(a0, a1, dinv16, eps)


# speedup vs baseline: 16.8756x; 1.0003x over previous
import jax
import jax.numpy as jnp
from jax import lax
from jax.experimental import pallas as pl
from jax.experimental.pallas import tpu as pltpu
from jax.experimental.pallas import tpu_sc as plsc

N = 10000
E = 160000
IN_DIMS = 256
HID1 = 256
HID2 = 128
OUT = 64

NC = 2
NS = 16
G = 125
NCH_D = E // (NC * NS * G)
NP = 10240
RPT = NP // NS
W128 = 128

_MESH = plsc.VectorSubcoreMesh(core_axis_name="c", subcore_axis_name="s")


NBUF = 2


def _pipelined_scatter_gather(table, out, src_v, dst_v, bufs, gsems, ssems,
                              acc, ngroups, r0, init_ref, init_sem):
    init_cp = pltpu.async_copy(
        init_ref.at[pl.ds(r0, RPT)], acc.at[pl.ds(r0, RPT)], init_sem)
    init_cp.wait()
    plsc.subcore_barrier()

    def group(i, carry):
        gcps = []
        for b in range(NBUF):
            j = i * NBUF + b
            gcps.append(
                pltpu.async_copy(table.at[src_v.at[j]], bufs[b], gsems[b]))
        scps = []
        for b in range(NBUF):
            j = i * NBUF + b
            gcps[b].wait()
            scps.append(
                pltpu.async_copy(bufs[b], acc.at[dst_v.at[j]], ssems[b],
                                 add=True))
        for b in range(NBUF):
            scps[b].wait()
        return carry

    lax.fori_loop(0, ngroups, group, 0)
    plsc.subcore_barrier()
    pltpu.sync_copy(acc.at[pl.ds(r0, RPT)], out.at[pl.ds(r0, RPT)])


def _deg_body(dst_hbm, ones_hbm, zeros_hbm, d0_hbm, d1_hbm, dst_v, ones_v,
              acc, init_sem, *ssems):
    c = lax.axis_index("c")
    s = lax.axis_index("s")
    w = c * NS + s
    pltpu.sync_copy(dst_hbm.at[w], dst_v)
    pltpu.sync_copy(ones_hbm, ones_v)
    r0 = s * RPT
    init_cp = pltpu.async_copy(
        zeros_hbm.at[pl.ds(r0, RPT)], acc.at[pl.ds(r0, RPT)], init_sem)
    init_cp.wait()
    plsc.subcore_barrier()

    def group(i, carry):
        scps = []
        for b in range(NBUF):
            j = i * NBUF + b
            scps.append(
                pltpu.async_copy(ones_v, acc.at[dst_v.at[j]], ssems[b],
                                 add=True))
        for b in range(NBUF):
            scps[b].wait()
        return carry

    lax.fori_loop(0, NCH_D // NBUF, group, 0)
    plsc.subcore_barrier()

    def wout(out):
        pltpu.sync_copy(acc.at[pl.ds(r0, RPT)], out.at[pl.ds(r0, RPT)])

    pl.when(c == 0)(lambda: wout(d0_hbm))
    pl.when(c == 1)(lambda: wout(d1_hbm))


_deg_call = pl.kernel(
    _deg_body,
    out_type=[jax.ShapeDtypeStruct((NP, W128), jnp.float32)] * 2,
    mesh=_MESH,
    scratch_types=[
        pltpu.VMEM((NCH_D, G), jnp.int32),
        pltpu.VMEM((G, W128), jnp.float32),
        pltpu.VMEM_SHARED((NP, W128), jnp.float32),
        pltpu.SemaphoreType.DMA,
    ] + [pltpu.SemaphoreType.DMA] * NBUF,
)


def _pass_edge_body(hws, zeros_hbm, src_hbm, dst_hbm, out0, out1,
                    src_v, dst_v, b0, b1, acc, init_sem, *sems):
    c = lax.axis_index("c")
    s = lax.axis_index("s")
    w = c * NS + s
    pltpu.sync_copy(src_hbm.at[w], src_v)
    pltpu.sync_copy(dst_hbm.at[w], dst_v)
    r0 = s * RPT
    bufs = (b0, b1)
    gsems, ssems = sems[:NBUF], sems[NBUF:]

    def work(init_ref, out):
        _pipelined_scatter_gather(hws, out, src_v, dst_v, bufs, gsems,
                                  ssems, acc, NCH_D // NBUF, r0,
                                  init_ref, init_sem)

    pl.when(c == 0)(lambda: work(hws, out0))
    pl.when(c == 1)(lambda: work(zeros_hbm, out1))


_pass_edge_call = pl.kernel(
    _pass_edge_body,
    out_type=[jax.ShapeDtypeStruct((NP, W128), jnp.float32)] * 2,
    mesh=_MESH,
    scratch_types=[
        pltpu.VMEM((NCH_D, G), jnp.int32),
        pltpu.VMEM((NCH_D, G), jnp.int32),
    ] + [pltpu.VMEM((G, W128), jnp.float32)] * NBUF + [
        pltpu.VMEM_SHARED((NP, W128), jnp.float32),
        pltpu.SemaphoreType.DMA,
    ] + [pltpu.SemaphoreType.DMA] * (2 * NBUF),
)


def _pass_edge2_body(hA, hB, zeros_hbm, src_hbm, dst_hbm,
                     outA0, outA1, outB0, outB1,
                     src_v, dst_v, b0, b1, acc, init_sem, *sems):
    c = lax.axis_index("c")
    s = lax.axis_index("s")
    w = c * NS + s
    pltpu.sync_copy(src_hbm.at[w], src_v)
    pltpu.sync_copy(dst_hbm.at[w], dst_v)
    r0 = s * RPT
    bufs = (b0, b1)
    gsems, ssems = sems[:NBUF], sems[NBUF:]

    def work(table, init_ref, out):
        _pipelined_scatter_gather(table, out, src_v, dst_v, bufs, gsems,
                                  ssems, acc, NCH_D // NBUF, r0,
                                  init_ref, init_sem)

    pl.when(c == 0)(lambda: work(hA, hA, outA0))
    pl.when(c == 1)(lambda: work(hA, zeros_hbm, outA1))
    plsc.subcore_barrier()
    pl.when(c == 0)(lambda: work(hB, hB, outB0))
    pl.when(c == 1)(lambda: work(hB, zeros_hbm, outB1))


_pass_edge2_call = pl.kernel(
    _pass_edge2_body,
    out_type=[jax.ShapeDtypeStruct((NP, W128), jnp.float32)] * 4,
    mesh=_MESH,
    scratch_types=[
        pltpu.VMEM((NCH_D, G), jnp.int32),
        pltpu.VMEM((NCH_D, G), jnp.int32),
    ] + [pltpu.VMEM((G, W128), jnp.float32)] * NBUF + [
        pltpu.VMEM_SHARED((NP, W128), jnp.float32),
        pltpu.SemaphoreType.DMA,
    ] + [pltpu.SemaphoreType.DMA] * (2 * NBUF),
)


TM = 1000


def _dinv_of(d0, d1):
    deg = d0[:, 0:1] + d1[:, 0:1] + 1.0
    return lax.rsqrt(deg)


def _mm1_body(x_ref, w_ref, b_ref, d0_ref, d1_ref, o0_ref, o1_ref, dv_ref):
    dinv = _dinv_of(d0_ref[...], d1_ref[...])
    hw = jnp.dot(x_ref[...], w_ref[...], preferred_element_type=jnp.float32)
    hws = (hw + b_ref[...]) * dinv
    half = hws.shape[1] // 2
    o0_ref[...] = hws[:, :half]
    o1_ref[...] = hws[:, half:]
    dv_ref[...] = jnp.broadcast_to(dinv, (dinv.shape[0], 16))


def _mm_cat_body(a0_ref, a1_ref, a2_ref, a3_ref, w_ref, b_ref, dv_ref,
                 o_ref):
    dinv = dv_ref[:, 0:1]
    h = jnp.concatenate([a0_ref[...] + a1_ref[...],
                         a2_ref[...] + a3_ref[...]], axis=1) * dinv
    h = jnp.maximum(h, 0.0)
    hw = jnp.dot(h, w_ref[...], preferred_element_type=jnp.float32)
    o_ref[...] = (hw + b_ref[...]) * dinv


def _mm_add_body(a0_ref, a1_ref, w_ref, b_ref, dv_ref, o_ref):
    dinv = dv_ref[:, 0:1]
    h = (a0_ref[...] + a1_ref[...]) * dinv
    h = jnp.maximum(h, 0.0)
    hw = jnp.dot(h, w_ref[...], preferred_element_type=jnp.float32)
    o_ref[...] = (hw + b_ref[...]) * dinv


def _z_body(a0_ref, a1_ref, dv_ref, eps_ref, z_ref):
    dinv = dv_ref[:, 0:1]
    ml = (a0_ref[...] + a1_ref[...]) * dinv
    mu = ml[:, :OUT]
    logstd = jnp.minimum(ml[:, OUT:], 10.0)
    z_ref[...] = mu + eps_ref[...] * jnp.exp(logstd)


def _row_spec(cols):
    return pl.BlockSpec((TM, cols), lambda i: (i, 0))


def _full_spec(shape):
    return pl.BlockSpec(shape, lambda i: (0,) * len(shape))


def _mm1(x, W, b, d0, d1):
    dout = W.shape[1]
    return pl.pallas_call(
        _mm1_body,
        grid=(N // TM,),
        in_specs=[
            _row_spec(x.shape[1]),
            _full_spec(W.shape),
            _full_spec((1, dout)),
            _row_spec(W128),
            _row_spec(W128),
        ],
        out_specs=[_row_spec(dout // 2), _row_spec(dout // 2),
                   _row_spec(16)],
        out_shape=[jax.ShapeDtypeStruct((NP, dout // 2), jnp.float32)] * 2
        + [jax.ShapeDtypeStruct((NP, 16), jnp.float32)],
    )(x, W, b.reshape(1, -1), d0, d1)


def _mm2(body, aparts, W, b, dinv16):
    dout = W.shape[1]
    return pl.pallas_call(
        body,
        grid=(N // TM,),
        in_specs=[_row_spec(a.shape[1]) for a in aparts] + [
            _full_spec(W.shape),
            _full_spec((1, dout)),
            _row_spec(16),
        ],
        out_specs=_row_spec(dout),
        out_shape=jax.ShapeDtypeStruct((NP, dout), jnp.float32),
    )(*aparts, W, b.reshape(1, -1), dinv16)


def _zfin(a0, a1, dinv16, eps):
    return pl.pallas_call(
        _z_body,
        grid=(N // TM,),
        in_specs=[
            _row_spec(W128),
            _row_spec(W128),
            _row_spec(16),
            _row_spec(OUT),
        ],
        out_specs=_row_spec(OUT),
        out_shape=jax.ShapeDtypeStruct((N, OUT), jnp.float32),
    )(a0, a1, dinv16, eps)


def kernel(x, edge_index, W1, b1, W2, b2, Wmu, bmu, Wls, bls):
    src = edge_index[0].astype(jnp.int32)
    dst = edge_index[1].astype(jnp.int32)
    src32 = src.reshape(NC * NS, NCH_D, G)
    dst32 = dst.reshape(NC * NS, NCH_D, G)

    ones_g = jnp.ones((G, W128), jnp.float32)
    zeros_n = jnp.zeros((NP, W128), jnp.float32)

    d0, d1 = _deg_call(dst32, ones_g, zeros_n)

    h0, h1, dinv16 = _mm1(x, W1, b1, d0, d1)
    p0a, p0b, p1a, p1b = _pass_edge2_call(h0, h1, zeros_n, src32, dst32)

    hws2 = _mm2(_mm_cat_body, (p0a, p0b, p1a, p1b), W2, b2, dinv16)
    a0, a1 = _pass_edge_call(hws2, zeros_n, src32, dst32)

    Wml = jnp.concatenate([Wmu, Wls], axis=1)
    bml = jnp.concatenate([bmu, bls], axis=0)
    hws3 = _mm2(_mm_add_body, (a0, a1), Wml, bml, dinv16)
    a0, a1 = _pass_edge_call(hws3, zeros_n, src32, dst32)

    eps = jax.random.normal(jax.random.key(42), (N, OUT), dtype=jnp.float32)
    return _zfin(a0, a1, dinv16, eps)
